# baseline (speedup 1.0000x reference)
import jax
import jax.numpy as jnp
from jax import lax
from jax.experimental import pallas as pl
from jax.experimental.pallas import tpu as pltpu
from jax.experimental.pallas import tpu_sc as plsc

D_MODEL = 64
NC, NS = 2, 16
NW = NC * NS
BB = 128
TT = 2


def _tec_body(x_hbm, tab_hbm, out_hbm,
              x_vA, x_vB, idx_vA, idx_vB, rows_vA, rows_vB,
              out_vA, out_vB, xsemA, xsemB, gsemA, gsemB, osemA, osemB):
    w = lax.axis_index("s") * NC + lax.axis_index("c")
    n_t = x_hbm.shape[0]
    n_stages = n_t // TT
    b0 = w * BB

    def x_copies(g, x_v, xsem):
        t0 = g * TT
        return [
            pltpu.make_async_copy(
                x_hbm.at[pl.ds(t0, TT), r, pl.ds(b0, BB)], x_v.at[r], xsem)
            for r in range(2)
        ]

    def gather_copies(idx_v, rows_v, gsem):
        return [
            pltpu.make_async_copy(
                tab_hbm.at[idx_v.at[j]], rows_v.at[pl.ds(j * BB, BB)], gsem)
            for j in range(2 * TT)
        ]

    def out_copies(g, out_v, osem):
        t0 = g * TT
        return [
            pltpu.make_async_copy(
                out_v.at[pl.ds((tt * 8 + dblk) * 8, 8)],
                out_hbm.at[t0 + tt, dblk, w], osem)
            for tt in range(TT)
            for dblk in range(D_MODEL // 8)
        ]

    def start(copies):
        for c in copies:
            c.start()

    def wait(copies):
        for c in copies:
            c.wait()

    def compute_idx(x_v, idx_v):
        for tt in range(TT):
            for r in range(2):
                for i in range(BB // 16):
                    xv = x_v[r, tt, pl.ds(i * 16, 16)]
                    iv = (xv * 5000.0 + 5000.0).astype(jnp.int32)
                    idx_v[2 * tt + r, pl.ds(i * 16, 16)] = iv

    def transpose_add(rows_v, out_v):
        for tt in range(TT):

            @pl.loop(0, BB, unroll=8)
            def _p(p):
                col = jnp.full((16,), p, jnp.int32)
                pr = tt * 2 * BB + p
                for d0 in range(0, D_MODEL, 16):
                    row = lax.iota(jnp.int32, 16) + (tt * D_MODEL + d0)
                    s = rows_v[pr, pl.ds(d0, 16)] + rows_v[pr + BB, pl.ds(d0, 16)]
                    plsc.store_scatter(out_v, [row, col], s)

    start(x_copies(0, x_vA, xsemA))
    start(x_copies(1, x_vB, xsemB))
    wait(x_copies(0, x_vA, xsemA))
    compute_idx(x_vA, idx_vA)
    start(gather_copies(idx_vA, rows_vA, gsemA))

    @pl.loop(0, n_stages // 2)
    def _iter(k):
        a = 2 * k
        wait(x_copies(a + 1, x_vB, xsemB))
        compute_idx(x_vB, idx_vB)
        start(gather_copies(idx_vB, rows_vB, gsemB))

        @pl.when(a + 2 < n_stages)
        def _():
            start(x_copies(a + 2, x_vA, xsemA))

        wait(gather_copies(idx_vA, rows_vA, gsemA))

        @pl.when(k >= 1)
        def _():
            wait(out_copies(a - 2, out_vA, osemA))

        transpose_add(rows_vA, out_vA)
        start(out_copies(a, out_vA, osemA))

        @pl.when(a + 2 < n_stages)
        def _():
            wait(x_copies(a + 2, x_vA, xsemA))
            compute_idx(x_vA, idx_vA)
            start(gather_copies(idx_vA, rows_vA, gsemA))
            start(x_copies(a + 3, x_vB, xsemB))

        wait(gather_copies(idx_vB, rows_vB, gsemB))

        @pl.when(k >= 1)
        def _():
            wait(out_copies(a - 1, out_vB, osemB))

        transpose_add(rows_vB, out_vB)
        start(out_copies(a + 1, out_vB, osemB))

    wait(out_copies(n_stages - 2, out_vA, osemA))
    wait(out_copies(n_stages - 1, out_vB, osemB))


def kernel(x, pos_enc):
    b, t, _ = x.shape
    xt = jnp.transpose(x, (1, 2, 0))

    mesh = plsc.VectorSubcoreMesh(
        core_axis_name="c", subcore_axis_name="s", num_cores=NC, num_subcores=NS
    )
    run = pl.kernel(
        _tec_body,
        out_type=jax.ShapeDtypeStruct((t, D_MODEL // 8, b // BB, 8, BB), jnp.float32),
        mesh=mesh,
        scratch_types=[
            pltpu.VMEM((2, TT, BB), jnp.float32),
            pltpu.VMEM((2, TT, BB), jnp.float32),
            pltpu.VMEM((2 * TT, BB), jnp.int32),
            pltpu.VMEM((2 * TT, BB), jnp.int32),
            pltpu.VMEM((2 * TT * BB, D_MODEL), jnp.float32),
            pltpu.VMEM((2 * TT * BB, D_MODEL), jnp.float32),
            pltpu.VMEM((TT * D_MODEL, BB), jnp.float32),
            pltpu.VMEM((TT * D_MODEL, BB), jnp.float32),
            pltpu.SemaphoreType.DMA,
            pltpu.SemaphoreType.DMA,
            pltpu.SemaphoreType.DMA,
            pltpu.SemaphoreType.DMA,
            pltpu.SemaphoreType.DMA,
            pltpu.SemaphoreType.DMA,
        ],
        compiler_params=pltpu.CompilerParams(
            use_tc_tiling_on_sc=False, needs_layout_passes=False
        ),
    )
    out5 = run(xt, pos_enc)
    o = jnp.transpose(out5, (0, 1, 3, 2, 4))
    o = o.reshape(t, D_MODEL, b)
    return jnp.transpose(o, (2, 0, 1))


# speedup vs baseline: 1.9268x; 1.9268x over previous
import jax
import jax.numpy as jnp
from jax import lax
from jax.experimental import pallas as pl
from jax.experimental.pallas import tpu as pltpu
from jax.experimental.pallas import tpu_sc as plsc

D_MODEL = 64
NC, NS = 2, 16
NW = NC * NS
BB = 128
TT = 2


def _tec_body(x_hbm, tab_hbm, out_hbm,
              x_vA, x_vB, idx_vA, idx_vB, rows_vA, rows_vB,
              out_vA, out_vB, xsemA, xsemB, gsemA, gsemB, osemA, osemB):
    w = lax.axis_index("s") * NC + lax.axis_index("c")
    n_t = x_hbm.shape[0]
    n_stages = n_t // TT
    b0 = w * BB

    def x_copies(g, x_v, xsem):
        t0 = g * TT
        return [
            pltpu.make_async_copy(
                x_hbm.at[pl.ds(t0, TT), r, pl.ds(b0, BB)], x_v.at[r], xsem)
            for r in range(2)
        ]

    def gather_copies(idx_v, rows_v, gsem):
        return [
            pltpu.make_async_copy(
                tab_hbm.at[idx_v.at[j]], rows_v.at[pl.ds(j * BB, BB)], gsem)
            for j in range(2 * TT)
        ]

    def out_copies(g, out_v, osem):
        t0 = g * TT
        return [
            pltpu.make_async_copy(
                out_v.at[pl.ds((tt * 8 + dblk) * 8, 8), pl.ds(0, BB)],
                out_hbm.at[t0 + tt, dblk, w], osem)
            for tt in range(TT)
            for dblk in range(D_MODEL // 8)
        ]

    def start(copies):
        for c in copies:
            c.start()

    def wait(copies):
        for c in copies:
            c.wait()

    def compute_idx(x_v, idx_v):
        for tt in range(TT):
            for r in range(2):
                for i in range(BB // 16):
                    xv = x_v[r, tt, pl.ds(i * 16, 16)]
                    iv = (xv * 5000.0 + 5000.0).astype(jnp.int32)
                    idx_v[2 * tt + r, pl.ds(i * 16, 16)] = iv

    def transpose_add(rows_v, out_v):
        for tt in range(TT):

            @pl.loop(0, BB, unroll=8)
            def _p(p):
                col = jnp.full((16,), p, jnp.int32)
                pr = tt * 2 * BB + p
                for d0 in range(0, D_MODEL, 16):
                    row = lax.iota(jnp.int32, 16) + (tt * D_MODEL + d0)
                    s = rows_v[pr, pl.ds(d0, 16)] + rows_v[pr + BB, pl.ds(d0, 16)]
                    plsc.store_scatter(out_v, [row, col], s)

    start(x_copies(0, x_vA, xsemA))
    start(x_copies(1, x_vB, xsemB))
    wait(x_copies(0, x_vA, xsemA))
    compute_idx(x_vA, idx_vA)
    start(gather_copies(idx_vA, rows_vA, gsemA))

    @pl.loop(0, n_stages // 2)
    def _iter(k):
        a = 2 * k
        wait(x_copies(a + 1, x_vB, xsemB))
        compute_idx(x_vB, idx_vB)
        start(gather_copies(idx_vB, rows_vB, gsemB))

        @pl.when(a + 2 < n_stages)
        def _():
            start(x_copies(a + 2, x_vA, xsemA))

        wait(gather_copies(idx_vA, rows_vA, gsemA))

        @pl.when(k >= 1)
        def _():
            wait(out_copies(a - 2, out_vA, osemA))

        transpose_add(rows_vA, out_vA)
        start(out_copies(a, out_vA, osemA))

        @pl.when(a + 2 < n_stages)
        def _():
            wait(x_copies(a + 2, x_vA, xsemA))
            compute_idx(x_vA, idx_vA)
            start(gather_copies(idx_vA, rows_vA, gsemA))
            start(x_copies(a + 3, x_vB, xsemB))

        wait(gather_copies(idx_vB, rows_vB, gsemB))

        @pl.when(k >= 1)
        def _():
            wait(out_copies(a - 1, out_vB, osemB))

        transpose_add(rows_vB, out_vB)
        start(out_copies(a + 1, out_vB, osemB))

    wait(out_copies(n_stages - 2, out_vA, osemA))
    wait(out_copies(n_stages - 1, out_vB, osemB))


def kernel(x, pos_enc):
    b, t, _ = x.shape
    xt = jnp.transpose(x, (1, 2, 0))

    mesh = plsc.VectorSubcoreMesh(
        core_axis_name="c", subcore_axis_name="s", num_cores=NC, num_subcores=NS
    )
    run = pl.kernel(
        _tec_body,
        out_type=jax.ShapeDtypeStruct((t, D_MODEL // 8, b // BB, 8, BB), jnp.float32),
        mesh=mesh,
        scratch_types=[
            pltpu.VMEM((2, TT, BB), jnp.float32),
            pltpu.VMEM((2, TT, BB), jnp.float32),
            pltpu.VMEM((2 * TT, BB), jnp.int32),
            pltpu.VMEM((2 * TT, BB), jnp.int32),
            pltpu.VMEM((2 * TT * BB, D_MODEL), jnp.float32),
            pltpu.VMEM((2 * TT * BB, D_MODEL), jnp.float32),
            pltpu.VMEM((TT * D_MODEL, BB + 1), jnp.float32),
            pltpu.VMEM((TT * D_MODEL, BB + 1), jnp.float32),
            pltpu.SemaphoreType.DMA,
            pltpu.SemaphoreType.DMA,
            pltpu.SemaphoreType.DMA,
            pltpu.SemaphoreType.DMA,
            pltpu.SemaphoreType.DMA,
            pltpu.SemaphoreType.DMA,
        ],
        compiler_params=pltpu.CompilerParams(
            use_tc_tiling_on_sc=False, needs_layout_passes=False
        ),
    )
    out5 = run(xt, pos_enc)
    o = jnp.transpose(out5, (0, 1, 3, 2, 4))
    o = o.reshape(t, D_MODEL, b)
    return jnp.transpose(o, (2, 0, 1))


# speedup vs baseline: 1.9369x; 1.0052x over previous
import jax
import jax.numpy as jnp
from jax import lax
from jax.experimental import pallas as pl
from jax.experimental.pallas import tpu as pltpu
from jax.experimental.pallas import tpu_sc as plsc

D_MODEL = 64
NC, NS = 2, 16
NW = NC * NS
BB = 128
TT = 2


def _tec_body(x_hbm, tab_hbm, out_hbm,
              x_vA, x_vB, idx_vA, idx_vB, rows_vA, rows_vB,
              out_vA, out_vB, xsemA, xsemB, gsemA, gsemB, osemA, osemB):
    w = lax.axis_index("s") * NC + lax.axis_index("c")
    n_t = x_hbm.shape[0]
    n_stages = n_t // TT
    b0 = w * BB

    def x_copies(g, x_v, xsem):
        t0 = g * TT
        return [
            pltpu.make_async_copy(
                x_hbm.at[pl.ds(t0, TT), r, pl.ds(b0, BB)], x_v.at[r], xsem)
            for r in range(2)
        ]

    def gather_copies(idx_v, rows_v, gsem):
        return [
            pltpu.make_async_copy(
                tab_hbm.at[idx_v.at[j]], rows_v.at[pl.ds(j * BB, BB)], gsem)
            for j in range(2 * TT)
        ]

    def out_copies(g, out_v, osem):
        t0 = g * TT
        return [
            pltpu.make_async_copy(
                out_v.at[pl.ds((tt * 8 + dblk) * 8, 8), pl.ds(0, BB)],
                out_hbm.at[t0 + tt, dblk, w], osem)
            for tt in range(TT)
            for dblk in range(D_MODEL // 8)
        ]

    def start(copies):
        for c in copies:
            c.start()

    def wait(copies):
        for c in copies:
            c.wait()

    def compute_idx(x_v, idx_v):
        for tt in range(TT):
            for r in range(2):
                for i in range(BB // 16):
                    xv = x_v[r, tt, pl.ds(i * 16, 16)]
                    iv = (xv * 5000.0 + 5000.0).astype(jnp.int32)
                    idx_v[2 * tt + r, pl.ds(i * 16, 16)] = iv

    def transpose_add(rows_v, out_v):
        for tt in range(TT):

            @pl.loop(0, BB, unroll=16)
            def _p(p):
                col = jnp.full((16,), p, jnp.int32)
                pr = tt * 2 * BB + p
                for d0 in range(0, D_MODEL, 16):
                    row = lax.iota(jnp.int32, 16) + (tt * D_MODEL + d0)
                    s = rows_v[pr, pl.ds(d0, 16)] + rows_v[pr + BB, pl.ds(d0, 16)]
                    plsc.store_scatter(out_v, [row, col], s)

    start(x_copies(0, x_vA, xsemA))
    start(x_copies(1, x_vB, xsemB))
    wait(x_copies(0, x_vA, xsemA))
    compute_idx(x_vA, idx_vA)
    start(x_copies(2, x_vA, xsemA))
    start(gather_copies(idx_vA, rows_vA, gsemA))

    @pl.loop(0, n_stages // 2)
    def _iter(k):
        a = 2 * k
        wait(x_copies(a + 1, x_vB, xsemB))
        compute_idx(x_vB, idx_vB)
        start(gather_copies(idx_vB, rows_vB, gsemB))

        @pl.when(a + 3 < n_stages)
        def _():
            start(x_copies(a + 3, x_vB, xsemB))

        wait(gather_copies(idx_vA, rows_vA, gsemA))

        @pl.when(k >= 1)
        def _():
            wait(out_copies(a - 2, out_vA, osemA))

        transpose_add(rows_vA, out_vA)
        start(out_copies(a, out_vA, osemA))

        @pl.when(a + 2 < n_stages)
        def _():
            wait(x_copies(a + 2, x_vA, xsemA))
            compute_idx(x_vA, idx_vA)
            start(gather_copies(idx_vA, rows_vA, gsemA))

            @pl.when(a + 4 < n_stages)
            def _():
                start(x_copies(a + 4, x_vA, xsemA))

        wait(gather_copies(idx_vB, rows_vB, gsemB))

        @pl.when(k >= 1)
        def _():
            wait(out_copies(a - 1, out_vB, osemB))

        transpose_add(rows_vB, out_vB)
        start(out_copies(a + 1, out_vB, osemB))

    wait(out_copies(n_stages - 2, out_vA, osemA))
    wait(out_copies(n_stages - 1, out_vB, osemB))


def kernel(x, pos_enc):
    b, t, _ = x.shape
    xt = jnp.transpose(x, (1, 2, 0))

    mesh = plsc.VectorSubcoreMesh(
        core_axis_name="c", subcore_axis_name="s", num_cores=NC, num_subcores=NS
    )
    run = pl.kernel(
        _tec_body,
        out_type=jax.ShapeDtypeStruct((t, D_MODEL // 8, b // BB, 8, BB), jnp.float32),
        mesh=mesh,
        scratch_types=[
            pltpu.VMEM((2, TT, BB), jnp.float32),
            pltpu.VMEM((2, TT, BB), jnp.float32),
            pltpu.VMEM((2 * TT, BB), jnp.int32),
            pltpu.VMEM((2 * TT, BB), jnp.int32),
            pltpu.VMEM((2 * TT * BB, D_MODEL), jnp.float32),
            pltpu.VMEM((2 * TT * BB, D_MODEL), jnp.float32),
            pltpu.VMEM((TT * D_MODEL, BB + 1), jnp.float32),
            pltpu.VMEM((TT * D_MODEL, BB + 1), jnp.float32),
            pltpu.SemaphoreType.DMA,
            pltpu.SemaphoreType.DMA,
            pltpu.SemaphoreType.DMA,
            pltpu.SemaphoreType.DMA,
            pltpu.SemaphoreType.DMA,
            pltpu.SemaphoreType.DMA,
        ],
        compiler_params=pltpu.CompilerParams(
            use_tc_tiling_on_sc=False, needs_layout_passes=False
        ),
    )
    out5 = run(xt, pos_enc)
    o = jnp.transpose(out5, (0, 1, 3, 2, 4))
    o = o.reshape(t, D_MODEL, b)
    return jnp.transpose(o, (2, 0, 1))


# speedup vs baseline: 3.7542x; 1.9383x over previous
import jax
import jax.numpy as jnp
from jax import lax
from jax.experimental import pallas as pl
from jax.experimental.pallas import tpu as pltpu
from jax.experimental.pallas import tpu_sc as plsc

D_MODEL = 64
NC, NS = 2, 16
NW = NC * NS
BB = 128
TT = 2


def _tec_body(x_hbm, tab_hbm, out_hbm,
              x_vA, x_vB, idx_vA, idx_vB, rows_vA, rows_vB,
              out_vA, out_vB, xsemA, xsemB, gsemA, gsemB, osemA, osemB):
    w = lax.axis_index("s") * NC + lax.axis_index("c")
    n_t = x_hbm.shape[0]
    n_stages = n_t // TT
    b0 = w * BB

    def x_copies(g, x_v, xsem):
        t0 = g * TT
        return [
            pltpu.make_async_copy(
                x_hbm.at[pl.ds(t0, TT), r, pl.ds(b0, BB)], x_v.at[r], xsem)
            for r in range(2)
        ]

    def gather_copies(idx_v, rows_v, gsem):
        return [
            pltpu.make_async_copy(
                tab_hbm.at[idx_v.at[j]], rows_v.at[pl.ds(j * BB, BB)], gsem)
            for j in range(2 * TT)
        ]

    def out_copies(g, out_v, osem):
        t0 = g * TT
        return [
            pltpu.make_async_copy(
                out_v.at[pl.ds((tt * 8 + dblk) * 8, 8), pl.ds(0, BB)],
                out_hbm.at[t0 + tt, dblk, w], osem)
            for tt in range(TT)
            for dblk in range(D_MODEL // 8)
        ]

    def start(copies):
        for c in copies:
            c.start()

    def wait(copies):
        for c in copies:
            c.wait()

    def compute_idx(x_v, idx_v):
        for tt in range(TT):
            for r in range(2):
                for i in range(BB // 16):
                    xv = x_v[r, tt, pl.ds(i * 16, 16)]
                    iv = (xv * 5000.0 + 5000.0).astype(jnp.int32)
                    idx_v[2 * tt + r, pl.ds(i * 16, 16)] = iv

    def transpose_add(rows_v, out_v):
        for tt in range(TT):

            @plsc.parallel_loop(0, BB, unroll=8)
            def _p(p):
                col = jnp.full((16,), p, jnp.int32)
                pr = tt * 2 * BB + p
                for d0 in range(0, D_MODEL, 16):
                    row = lax.iota(jnp.int32, 16) + (tt * D_MODEL + d0)
                    s = rows_v[pr, pl.ds(d0, 16)] + rows_v[pr + BB, pl.ds(d0, 16)]
                    plsc.store_scatter(out_v, [row, col], s)

    start(x_copies(0, x_vA, xsemA))
    start(x_copies(1, x_vB, xsemB))
    wait(x_copies(0, x_vA, xsemA))
    compute_idx(x_vA, idx_vA)
    start(gather_copies(idx_vA, rows_vA, gsemA))

    @pl.loop(0, n_stages // 2)
    def _iter(k):
        a = 2 * k
        wait(x_copies(a + 1, x_vB, xsemB))
        compute_idx(x_vB, idx_vB)
        start(gather_copies(idx_vB, rows_vB, gsemB))

        @pl.when(a + 2 < n_stages)
        def _():
            start(x_copies(a + 2, x_vA, xsemA))

        wait(gather_copies(idx_vA, rows_vA, gsemA))

        @pl.when(k >= 1)
        def _():
            wait(out_copies(a - 2, out_vA, osemA))

        transpose_add(rows_vA, out_vA)
        start(out_copies(a, out_vA, osemA))

        @pl.when(a + 2 < n_stages)
        def _():
            wait(x_copies(a + 2, x_vA, xsemA))
            compute_idx(x_vA, idx_vA)
            start(gather_copies(idx_vA, rows_vA, gsemA))
            start(x_copies(a + 3, x_vB, xsemB))

        wait(gather_copies(idx_vB, rows_vB, gsemB))

        @pl.when(k >= 1)
        def _():
            wait(out_copies(a - 1, out_vB, osemB))

        transpose_add(rows_vB, out_vB)
        start(out_copies(a + 1, out_vB, osemB))

    wait(out_copies(n_stages - 2, out_vA, osemA))
    wait(out_copies(n_stages - 1, out_vB, osemB))


def kernel(x, pos_enc):
    b, t, _ = x.shape
    xt = jnp.transpose(x, (1, 2, 0))

    mesh = plsc.VectorSubcoreMesh(
        core_axis_name="c", subcore_axis_name="s", num_cores=NC, num_subcores=NS
    )
    run = pl.kernel(
        _tec_body,
        out_type=jax.ShapeDtypeStruct((t, D_MODEL // 8, b // BB, 8, BB), jnp.float32),
        mesh=mesh,
        scratch_types=[
            pltpu.VMEM((2, TT, BB), jnp.float32),
            pltpu.VMEM((2, TT, BB), jnp.float32),
            pltpu.VMEM((2 * TT, BB), jnp.int32),
            pltpu.VMEM((2 * TT, BB), jnp.int32),
            pltpu.VMEM((2 * TT * BB, D_MODEL), jnp.float32),
            pltpu.VMEM((2 * TT * BB, D_MODEL), jnp.float32),
            pltpu.VMEM((TT * D_MODEL, BB + 1), jnp.float32),
            pltpu.VMEM((TT * D_MODEL, BB + 1), jnp.float32),
            pltpu.SemaphoreType.DMA,
            pltpu.SemaphoreType.DMA,
            pltpu.SemaphoreType.DMA,
            pltpu.SemaphoreType.DMA,
            pltpu.SemaphoreType.DMA,
            pltpu.SemaphoreType.DMA,
        ],
        compiler_params=pltpu.CompilerParams(
            use_tc_tiling_on_sc=False, needs_layout_passes=False
        ),
    )
    out5 = run(xt, pos_enc)
    o = jnp.transpose(out5, (0, 1, 3, 2, 4))
    o = o.reshape(t, D_MODEL, b)
    return jnp.transpose(o, (2, 0, 1))
